# Initial kernel scaffold; baseline (speedup 1.0000x reference)
import math

import jax
import jax.numpy as jnp
import numpy as np
from jax import lax
from jax.experimental import pallas as pl
from jax.experimental.pallas import tpu as pltpu
from jax.experimental.pallas import tpu_sc as plsc

VOCAB = 100000
EMBED = 128
SEQ = 200
BATCH = 1024
HALF = 100
NC, NS = 2, 16
NW = NC * NS
SEQ_PER_W = BATCH // NW
CH_PER_W = SEQ_PER_W * 2


def _pe_table():
    pos = np.arange(SEQ, dtype=np.float32)[:, None]
    div = np.exp(
        np.arange(0, EMBED, 2, dtype=np.float32) * -(math.log(10000.0) / EMBED)
    )
    pe = np.zeros((SEQ, EMBED), dtype=np.float32)
    pe[:, 0::2] = np.sin(pos * div)
    pe[:, 1::2] = np.cos(pos * div)
    return pe


_PE = _pe_table()


def _body(idx_hbm, table_hbm, pe_hbm, out_hbm, idx_v, pe_v, rows_v, gsem):
    wid = lax.axis_index("s") * NC + lax.axis_index("c")
    pltpu.sync_copy(idx_hbm.at[pl.ds(wid * CH_PER_W, CH_PER_W)], idx_v)
    pltpu.sync_copy(pe_hbm, pe_v)
    row0 = wid * SEQ_PER_W * SEQ

    def seq_body(s, _):
        for h in range(2):
            c = s * 2 + h
            pltpu.async_copy(table_hbm.at[idx_v.at[c]], rows_v, gsem).wait()

            def add_row(r, _):
                for j in range(8):
                    sl = pl.ds(j * 16, 16)
                    rows_v[r, sl] = rows_v[r, sl] + pe_v[h * HALF + r, sl]
                return 0

            lax.fori_loop(0, HALF, add_row, 0)
            pltpu.sync_copy(rows_v, out_hbm.at[pl.ds(row0 + c * HALF, HALF)])
        return 0

    lax.fori_loop(0, SEQ_PER_W, seq_body, 0)


def kernel(token_seq, token_table):
    idx = token_seq.astype(jnp.int32).reshape(BATCH * 2, HALF)
    pe = jnp.asarray(_PE)
    f = pl.kernel(
        _body,
        out_type=jax.ShapeDtypeStruct((BATCH * SEQ, EMBED), jnp.float32),
        mesh=plsc.VectorSubcoreMesh(core_axis_name="c", subcore_axis_name="s"),
        scratch_types=[
            pltpu.VMEM((CH_PER_W, HALF), jnp.int32),
            pltpu.VMEM((SEQ, EMBED), jnp.float32),
            pltpu.VMEM((HALF, EMBED), jnp.float32),
            pltpu.SemaphoreType.DMA,
        ],
    )
    out = f(idx, token_table, pe)
    return out.reshape(BATCH, SEQ, EMBED)


# speedup vs baseline: 3.8141x; 3.8141x over previous
import math

import jax
import jax.numpy as jnp
import numpy as np
from jax import lax
from jax.experimental import pallas as pl
from jax.experimental.pallas import tpu as pltpu
from jax.experimental.pallas import tpu_sc as plsc

VOCAB = 100000
EMBED = 128
SEQ = 200
BATCH = 1024
HALF = 100
NC, NS = 2, 16
NW = NC * NS
SEQ_PER_W = BATCH // NW
CH_PER_W = SEQ_PER_W * 2


def _pe_table():
    pos = np.arange(SEQ, dtype=np.float32)[:, None]
    div = np.exp(
        np.arange(0, EMBED, 2, dtype=np.float32) * -(math.log(10000.0) / EMBED)
    )
    pe = np.zeros((SEQ, EMBED), dtype=np.float32)
    pe[:, 0::2] = np.sin(pos * div)
    pe[:, 1::2] = np.cos(pos * div)
    return pe


_PE = _pe_table()


def _body(idx_hbm, table_hbm, pe_hbm, out_hbm, idx_v, pe_v, rows_v, gsem):
    wid = lax.axis_index("s") * NC + lax.axis_index("c")
    pltpu.sync_copy(idx_hbm.at[pl.ds(wid * CH_PER_W, CH_PER_W)], idx_v)
    pltpu.sync_copy(pe_hbm, pe_v)
    row0 = wid * SEQ_PER_W * SEQ

    def seq_body(s, _):
        for h in range(2):
            pltpu.async_copy(
                table_hbm.at[idx_v.at[s * 2 + h]],
                rows_v.at[pl.ds(h * HALF, HALF)],
                gsem,
            ).wait()

        def add_row(r, _):
            for j in range(8):
                sl = pl.ds(j * 16, 16)
                rows_v[r, sl] = rows_v[r, sl] + pe_v[r, sl]
            return 0

        lax.fori_loop(0, SEQ, add_row, 0)
        pltpu.sync_copy(rows_v, out_hbm.at[pl.ds(row0 + s * SEQ, SEQ)])
        return 0

    lax.fori_loop(0, SEQ_PER_W, seq_body, 0)


def kernel(token_seq, token_table):
    idx = token_seq.astype(jnp.int32).reshape(BATCH * 2, HALF)
    pe = jnp.asarray(_PE)
    f = pl.kernel(
        _body,
        out_type=jax.ShapeDtypeStruct((BATCH * SEQ, EMBED), jnp.float32),
        mesh=plsc.VectorSubcoreMesh(core_axis_name="c", subcore_axis_name="s"),
        scratch_types=[
            pltpu.VMEM((CH_PER_W, HALF), jnp.int32),
            pltpu.VMEM((SEQ, EMBED), jnp.float32),
            pltpu.VMEM((SEQ, EMBED), jnp.float32),
            pltpu.SemaphoreType.DMA,
        ],
    )
    out = f(idx, token_table, pe)
    return out.reshape(BATCH, SEQ, EMBED)
